# baseline (speedup 1.0000x reference)
import dataclasses
import functools

import jax
import jax.numpy as jnp
from jax import lax
from jax.experimental import pallas as pl
from jax.experimental.pallas import tpu as pltpu
from jax.experimental.pallas import tpu_sc as plsc

F32 = jnp.float32


def _tree_reduce(op, xs):
    xs = list(xs)
    while len(xs) > 1:
        nxt = [op(xs[i], xs[i + 1]) for i in range(0, len(xs) - 1, 2)]
        if len(xs) % 2:
            nxt.append(xs[-1])
        xs = nxt
    return xs[0]


def _scores(emb, u_row):
    N, D = emb.shape
    BLK = 2000
    grid = N // BLK

    def body(e_ref, u_ref, o_ref):
        o_ref[...] = jnp.sum(e_ref[...] * u_ref[...], axis=1)[None, None, :]

    out = pl.pallas_call(
        body,
        grid=(grid,),
        in_specs=[
            pl.BlockSpec((BLK, D), lambda i: (i, 0)),
            pl.BlockSpec((1, D), lambda i: (0, 0)),
        ],
        out_specs=pl.BlockSpec((1, 1, BLK), lambda i: (i, 0, 0)),
        out_shape=jax.ShapeDtypeStruct((grid, 1, BLK), F32),
    )(emb, u_row)
    return out.reshape(N)


def _sc_agg(adj_flat, adj_t_flat, s, emb, F_PAD, FW, K, D):
    NW = 32
    CH = 4
    GCH = 128
    mesh = plsc.VectorSubcoreMesh(core_axis_name="c", subcore_axis_name="s")
    NLANE = 16
    NSUB = D // NLANE
    cp = pltpu.CompilerParams()
    if "needs_layout_passes" in pltpu.CompilerParams.__dataclass_fields__:
        cp = dataclasses.replace(cp, needs_layout_passes=False)
    if "use_tc_tiling_on_sc" in pltpu.CompilerParams.__dataclass_fields__:
        cp = dataclasses.replace(cp, use_tc_tiling_on_sc=False)

    NBUF = 4

    BF16 = jnp.bfloat16
    NCH = FW // CH
    DW = D // 2

    @functools.partial(
        pl.kernel,
        out_type=jax.ShapeDtypeStruct((F_PAD, DW), jnp.int32),
        mesh=mesh,
        compiler_params=cp,
        scratch_types=[
            pltpu.VMEM((FW * K,), jnp.int32),
            pltpu.VMEM((FW * K,), jnp.int32),
            pltpu.VMEM((FW * K,), F32),
            pltpu.VMEM((FW * K,), F32),
            pltpu.VMEM((NBUF, CH * K, DW), jnp.int32),
            pltpu.VMEM((FW, DW), jnp.int32),
            pltpu.SemaphoreType.DMA,
            pltpu.SemaphoreType.DMA((NBUF,)),
        ],
    )
    def kern(adj_f_hbm, adj_t_hbm, s_hbm, emb_hbm, agg_hbm,
             adj_v, adjt_v, sg_v, w_v, rows_v, out_v,
             sem_m, sem_g):
        cid = lax.axis_index("c")
        sid = lax.axis_index("s")
        wid = sid * 2 + cid
        base_f = wid * FW

        pltpu.async_copy(adj_f_hbm.at[pl.ds(base_f * K, FW * K)], adj_v,
                         sem_m)

        @pl.loop(0, K)
        def _adjt(k):
            pltpu.async_copy(
                adj_t_hbm.at[pl.ds(k * F_PAD + base_f, FW)],
                adjt_v.at[pl.ds(k * FW, FW)],
                sem_m,
            )

        pltpu.make_async_copy(adj_f_hbm.at[pl.ds(0, FW * K)], adj_v,
                              sem_m).wait()
        pltpu.make_async_copy(adj_t_hbm.at[pl.ds(0, FW * K)], adjt_v,
                              sem_m).wait()

        def _start_gather(ch, j):
            pltpu.async_copy(
                emb_hbm.at[adj_v.at[pl.ds(ch * (CH * K), CH * K)]],
                rows_v.at[j], sem_g.at[j])

        for j in range(NBUF):
            _start_gather(j, j)

        @pl.loop(0, (FW * K) // GCH)
        def _sg(c):
            pltpu.async_copy(
                s_hbm.at[adjt_v.at[pl.ds(c * GCH, GCH)]],
                sg_v.at[pl.ds(c * GCH, GCH)],
                sem_m,
            )

        pltpu.make_async_copy(s_hbm.at[pl.ds(0, FW * K)], sg_v,
                              sem_m).wait()

        @pl.loop(0, FW // NLANE)
        def _smax(g):
            logits = []
            for k in range(K):
                off = k * FW + g * NLANE
                a = adjt_v[pl.ds(off, NLANE)]
                sv = sg_v[pl.ds(off, NLANE)]
                logits.append(sv + jnp.where(a != 0, 0.0, -10000.0))
            mx = _tree_reduce(jnp.maximum, logits)
            es = [jnp.exp(l - mx) for l in logits]
            tot = _tree_reduce(jnp.add, es)
            inv = 1.0 / tot
            fidx = (lax.iota(jnp.int32, NLANE) + g * NLANE) * K
            for k in range(K):
                plsc.store_scatter(w_v, [fidx + k], es[k] * inv)

        NPAIR = DW // NLANE

        @pl.loop(0, NCH, step=NBUF)
        def _acc(c0):
            for j in range(NBUF):
                ch = c0 + j
                f0 = ch * CH
                pltpu.make_async_copy(
                    emb_hbm.at[adj_v.at[pl.ds(0, CH * K)]],
                    rows_v.at[j], sem_g.at[j]).wait()

                for i in range(CH):
                    wva = w_v[pl.ds((f0 + i) * K, NLANE)]
                    wvb = w_v[pl.ds((f0 + i) * K + NLANE, NLANE)]
                    acc_a = [None] * NPAIR
                    acc_b = [None] * NPAIR
                    for k in range(K):
                        wk = wva[k] if k < NLANE else wvb[k - NLANE]
                        for c in range(NPAIR):
                            pk = plsc.bitcast(
                                rows_v[j, i * K + k,
                                       pl.ds(c * NLANE, NLANE)], BF16)
                            a, b = plsc.unpack(
                                pk, format=plsc.PackFormat.INTERLEAVED)
                            if k == 0:
                                acc_a[c] = wk * a
                                acc_b[c] = wk * b
                            else:
                                acc_a[c] = acc_a[c] + wk * a
                                acc_b[c] = acc_b[c] + wk * b
                    for c in range(NPAIR):
                        out_v[f0 + i, pl.ds(c * NLANE, NLANE)] = (
                            plsc.bitcast(
                                plsc.pack(acc_a[c], acc_b[c],
                                          format=plsc.PackFormat.INTERLEAVED),
                                jnp.int32))

                @pl.when(ch + NBUF < NCH)
                def _next_gather():
                    _start_gather(ch + NBUF, j)

        pltpu.sync_copy(out_v, agg_hbm.at[pl.ds(base_f, FW)])

    return kern(adj_flat, adj_t_flat, s, emb)


def _gate(ef, ag, w1t, w2t, b_row):
    F, D = ef.shape
    BLK = 2000

    def body(ef_ref, ag_ref, w1_ref, w2_ref, b_ref, o_ref):
        e = ef_ref[...]
        a = ag_ref[...].astype(F32)
        g = (jnp.dot(e, w1_ref[...], preferred_element_type=F32)
             + jnp.dot(a, w2_ref[...], preferred_element_type=F32)
             + b_ref[...])
        g = jax.nn.sigmoid(g)
        o_ref[...] = g * e + (1.0 - g) * a

    return pl.pallas_call(
        body,
        grid=(F // BLK,),
        in_specs=[
            pl.BlockSpec((BLK, D), lambda i: (i, 0)),
            pl.BlockSpec((BLK, D), lambda i: (i, 0)),
            pl.BlockSpec((D, D), lambda i: (0, 0)),
            pl.BlockSpec((D, D), lambda i: (0, 0)),
            pl.BlockSpec((1, D), lambda i: (0, 0)),
        ],
        out_specs=pl.BlockSpec((BLK, D), lambda i: (i, 0)),
        out_shape=jax.ShapeDtypeStruct((F, D), F32),
    )(ef, ag, w1t, w2t, b_row)


def kernel(adjacency_fi, embedding_i, emb_f_weight, u, W_w, W_b):
    F, K = adjacency_fi.shape
    N, D = embedding_i.shape
    NW = 32
    FW = ((F + NW - 1) // NW + 15) // 16 * 16
    F_PAD = FW * NW

    adj = adjacency_fi.astype(jnp.int32)
    adj = jnp.pad(adj, ((0, F_PAD - F), (0, 0)))
    adj_flat = adj.reshape(-1)
    adj_t_flat = adj.T.reshape(-1)

    s = _scores(embedding_i, u.reshape(1, D))
    emb16 = embedding_i.astype(jnp.bfloat16)
    emb_pk = jax.lax.bitcast_convert_type(
        emb16.reshape(N, D // 2, 2), jnp.int32)
    agg_pk = _sc_agg(adj_flat, adj_t_flat, s, emb_pk, F_PAD, FW, K, D)
    agg = jax.lax.bitcast_convert_type(
        agg_pk, jnp.bfloat16).reshape(F_PAD, D)[:F]
    w1t = W_w[:, :D].T
    w2t = W_w[:, D:].T
    return _gate(emb_f_weight, agg, w1t, w2t, W_b.reshape(1, D))


# speedup vs baseline: 2.2938x; 2.2938x over previous
import dataclasses
import functools

import jax
import jax.numpy as jnp
from jax import lax
from jax.experimental import pallas as pl
from jax.experimental.pallas import tpu as pltpu
from jax.experimental.pallas import tpu_sc as plsc

F32 = jnp.float32


def _tree_reduce(op, xs):
    xs = list(xs)
    while len(xs) > 1:
        nxt = [op(xs[i], xs[i + 1]) for i in range(0, len(xs) - 1, 2)]
        if len(xs) % 2:
            nxt.append(xs[-1])
        xs = nxt
    return xs[0]


def _scores(emb, u_row):
    N, D = emb.shape
    BLK = 2000
    grid = N // BLK

    def body(e_ref, u_ref, o_ref):
        o_ref[...] = jnp.sum(e_ref[...] * u_ref[...], axis=1)[None, None, :]

    out = pl.pallas_call(
        body,
        grid=(grid,),
        in_specs=[
            pl.BlockSpec((BLK, D), lambda i: (i, 0)),
            pl.BlockSpec((1, D), lambda i: (0, 0)),
        ],
        out_specs=pl.BlockSpec((1, 1, BLK), lambda i: (i, 0, 0)),
        out_shape=jax.ShapeDtypeStruct((grid, 1, BLK), F32),
    )(emb, u_row)
    return out.reshape(N)


def _pack_table(emb_flat, N, D):
    NW = 32
    DW = D // 2
    ROWS_T = N // NW
    RCH = 125
    NCHP = ROWS_T // RCH
    NBP = 5
    mesh = plsc.VectorSubcoreMesh(core_axis_name="c", subcore_axis_name="s")
    cp = pltpu.CompilerParams()
    if "needs_layout_passes" in pltpu.CompilerParams.__dataclass_fields__:
        cp = dataclasses.replace(cp, needs_layout_passes=False)
    if "use_tc_tiling_on_sc" in pltpu.CompilerParams.__dataclass_fields__:
        cp = dataclasses.replace(cp, use_tc_tiling_on_sc=False)

    @functools.partial(
        pl.kernel,
        out_type=jax.ShapeDtypeStruct((N, DW), jnp.int32),
        mesh=mesh,
        compiler_params=cp,
        scratch_types=[
            pltpu.VMEM((NBP, RCH * D), F32),
            pltpu.VMEM((NBP, RCH, DW), jnp.int32),
            pltpu.SemaphoreType.DMA((NBP,)),
            pltpu.SemaphoreType.DMA((NBP,)),
        ],
    )
    def kern(src_hbm, dst_hbm, in_v, out_v, sem_i, sem_o):
        cid = lax.axis_index("c")
        sid = lax.axis_index("s")
        wid = sid * 2 + cid
        r0 = wid * ROWS_T
        ev = 2 * lax.iota(jnp.int32, 16)

        def _start_in(ch, j):
            pltpu.async_copy(
                src_hbm.at[pl.ds((r0 + ch * RCH) * D, RCH * D)],
                in_v.at[j], sem_i.at[j])

        for j in range(NBP):
            _start_in(j, j)

        @pl.loop(0, NCHP, step=NBP)
        def _go(c0):
            for j in range(NBP):
                ch = c0 + j
                pltpu.make_async_copy(src_hbm.at[pl.ds(0, RCH * D)],
                                      in_v.at[j], sem_i.at[j]).wait()

                @pl.when(c0 > 0)
                def _wait_out():
                    pltpu.make_async_copy(out_v.at[j],
                                          dst_hbm.at[pl.ds(0, RCH)],
                                          sem_o.at[j]).wait()

                @pl.loop(0, RCH)
                def _row(r):
                    for c in range(D // 32):
                        base = r * D + c * 32
                        a = plsc.load_gather(in_v.at[j], [base + ev])
                        b = plsc.load_gather(in_v.at[j], [base + ev + 1])
                        out_v[j, r, pl.ds(c * 16, 16)] = plsc.bitcast(
                            plsc.pack(a, b,
                                      format=plsc.PackFormat.INTERLEAVED),
                            jnp.int32)

                pltpu.async_copy(out_v.at[j],
                                 dst_hbm.at[pl.ds(r0 + ch * RCH, RCH)],
                                 sem_o.at[j])

                @pl.when(ch + NBP < NCHP)
                def _next_in():
                    _start_in(ch + NBP, j)

        for j in range(NBP):
            pltpu.make_async_copy(out_v.at[j], dst_hbm.at[pl.ds(0, RCH)],
                                  sem_o.at[j]).wait()

    return kern(emb_flat)


def _sc_agg(adj_flat, adj_t_flat, s, emb, F_PAD, FW, K, D):
    NW = 32
    CH = 4
    GCH = 128
    mesh = plsc.VectorSubcoreMesh(core_axis_name="c", subcore_axis_name="s")
    NLANE = 16
    NSUB = D // NLANE
    cp = pltpu.CompilerParams()
    if "needs_layout_passes" in pltpu.CompilerParams.__dataclass_fields__:
        cp = dataclasses.replace(cp, needs_layout_passes=False)
    if "use_tc_tiling_on_sc" in pltpu.CompilerParams.__dataclass_fields__:
        cp = dataclasses.replace(cp, use_tc_tiling_on_sc=False)

    NBUF = 4

    BF16 = jnp.bfloat16
    NCH = FW // CH
    DW = D // 2

    @functools.partial(
        pl.kernel,
        out_type=jax.ShapeDtypeStruct((F_PAD, DW), jnp.int32),
        mesh=mesh,
        compiler_params=cp,
        scratch_types=[
            pltpu.VMEM((FW * K,), jnp.int32),
            pltpu.VMEM((FW * K,), jnp.int32),
            pltpu.VMEM((FW * K,), F32),
            pltpu.VMEM((FW * K,), F32),
            pltpu.VMEM((NBUF, CH * K, DW), jnp.int32),
            pltpu.VMEM((FW, DW), jnp.int32),
            pltpu.SemaphoreType.DMA,
            pltpu.SemaphoreType.DMA((NBUF,)),
        ],
    )
    def kern(adj_f_hbm, adj_t_hbm, s_hbm, emb_hbm, agg_hbm,
             adj_v, adjt_v, sg_v, w_v, rows_v, out_v,
             sem_m, sem_g):
        cid = lax.axis_index("c")
        sid = lax.axis_index("s")
        wid = sid * 2 + cid
        base_f = wid * FW

        pltpu.async_copy(adj_f_hbm.at[pl.ds(base_f * K, FW * K)], adj_v,
                         sem_m)

        @pl.loop(0, K)
        def _adjt(k):
            pltpu.async_copy(
                adj_t_hbm.at[pl.ds(k * F_PAD + base_f, FW)],
                adjt_v.at[pl.ds(k * FW, FW)],
                sem_m,
            )

        pltpu.make_async_copy(adj_f_hbm.at[pl.ds(0, FW * K)], adj_v,
                              sem_m).wait()
        pltpu.make_async_copy(adj_t_hbm.at[pl.ds(0, FW * K)], adjt_v,
                              sem_m).wait()

        def _start_gather(ch, j):
            pltpu.async_copy(
                emb_hbm.at[adj_v.at[pl.ds(ch * (CH * K), CH * K)]],
                rows_v.at[j], sem_g.at[j])

        for j in range(NBUF):
            _start_gather(j, j)

        @pl.loop(0, (FW * K) // GCH)
        def _sg(c):
            pltpu.async_copy(
                s_hbm.at[adjt_v.at[pl.ds(c * GCH, GCH)]],
                sg_v.at[pl.ds(c * GCH, GCH)],
                sem_m,
            )

        pltpu.make_async_copy(s_hbm.at[pl.ds(0, FW * K)], sg_v,
                              sem_m).wait()

        @pl.loop(0, FW // NLANE)
        def _smax(g):
            logits = []
            for k in range(K):
                off = k * FW + g * NLANE
                a = adjt_v[pl.ds(off, NLANE)]
                sv = sg_v[pl.ds(off, NLANE)]
                logits.append(sv + jnp.where(a != 0, 0.0, -10000.0))
            mx = _tree_reduce(jnp.maximum, logits)
            es = [jnp.exp(l - mx) for l in logits]
            tot = _tree_reduce(jnp.add, es)
            inv = 1.0 / tot
            fidx = (lax.iota(jnp.int32, NLANE) + g * NLANE) * K
            for k in range(K):
                plsc.store_scatter(w_v, [fidx + k], es[k] * inv)

        NPAIR = DW // NLANE

        @pl.loop(0, NCH, step=NBUF)
        def _acc(c0):
            for j in range(NBUF):
                ch = c0 + j
                f0 = ch * CH
                pltpu.make_async_copy(
                    emb_hbm.at[adj_v.at[pl.ds(0, CH * K)]],
                    rows_v.at[j], sem_g.at[j]).wait()

                for i in range(CH):
                    wva = w_v[pl.ds((f0 + i) * K, NLANE)]
                    wvb = w_v[pl.ds((f0 + i) * K + NLANE, NLANE)]
                    acc_a = [None] * NPAIR
                    acc_b = [None] * NPAIR
                    for k in range(K):
                        wk = wva[k] if k < NLANE else wvb[k - NLANE]
                        for c in range(NPAIR):
                            pk = plsc.bitcast(
                                rows_v[j, i * K + k,
                                       pl.ds(c * NLANE, NLANE)], BF16)
                            a, b = plsc.unpack(
                                pk, format=plsc.PackFormat.INTERLEAVED)
                            if k == 0:
                                acc_a[c] = wk * a
                                acc_b[c] = wk * b
                            else:
                                acc_a[c] = acc_a[c] + wk * a
                                acc_b[c] = acc_b[c] + wk * b
                    for c in range(NPAIR):
                        out_v[f0 + i, pl.ds(c * NLANE, NLANE)] = (
                            plsc.bitcast(
                                plsc.pack(acc_a[c], acc_b[c],
                                          format=plsc.PackFormat.INTERLEAVED),
                                jnp.int32))

                @pl.when(ch + NBUF < NCH)
                def _next_gather():
                    _start_gather(ch + NBUF, j)

        pltpu.sync_copy(out_v, agg_hbm.at[pl.ds(base_f, FW)])

    return kern(adj_flat, adj_t_flat, s, emb)


def _gate(ef, ag, w1t, w2t, b_row):
    F, D = ef.shape
    BLK = 2000

    def body(ef_ref, ag_ref, w1_ref, w2_ref, b_ref, o_ref):
        e = ef_ref[...]
        a = ag_ref[...].astype(F32)
        g = (jnp.dot(e, w1_ref[...], preferred_element_type=F32)
             + jnp.dot(a, w2_ref[...], preferred_element_type=F32)
             + b_ref[...])
        g = jax.nn.sigmoid(g)
        o_ref[...] = g * e + (1.0 - g) * a

    return pl.pallas_call(
        body,
        grid=(F // BLK,),
        in_specs=[
            pl.BlockSpec((BLK, D), lambda i: (i, 0)),
            pl.BlockSpec((BLK, D), lambda i: (i, 0)),
            pl.BlockSpec((D, D), lambda i: (0, 0)),
            pl.BlockSpec((D, D), lambda i: (0, 0)),
            pl.BlockSpec((1, D), lambda i: (0, 0)),
        ],
        out_specs=pl.BlockSpec((BLK, D), lambda i: (i, 0)),
        out_shape=jax.ShapeDtypeStruct((F, D), F32),
    )(ef, ag, w1t, w2t, b_row)


def kernel(adjacency_fi, embedding_i, emb_f_weight, u, W_w, W_b):
    F, K = adjacency_fi.shape
    N, D = embedding_i.shape
    NW = 32
    FW = ((F + NW - 1) // NW + 15) // 16 * 16
    F_PAD = FW * NW

    adj = adjacency_fi.astype(jnp.int32)
    adj = jnp.pad(adj, ((0, F_PAD - F), (0, 0)))
    adj_flat = adj.reshape(-1)
    adj_t_flat = adj.T.reshape(-1)

    s = _scores(embedding_i, u.reshape(1, D))
    emb_pk = _pack_table(embedding_i.reshape(N * D), N, D)
    agg_pk = _sc_agg(adj_flat, adj_t_flat, s, emb_pk, F_PAD, FW, K, D)
    agg = jax.lax.bitcast_convert_type(
        agg_pk, jnp.bfloat16).reshape(F_PAD, D)[:F]
    w1t = W_w[:, :D].T
    w2t = W_w[:, D:].T
    return _gate(emb_f_weight, agg, w1t, w2t, W_b.reshape(1, D))


# speedup vs baseline: 2.4205x; 1.0552x over previous
import dataclasses
import functools

import jax
import jax.numpy as jnp
from jax import lax
from jax.experimental import pallas as pl
from jax.experimental.pallas import tpu as pltpu
from jax.experimental.pallas import tpu_sc as plsc

F32 = jnp.float32


def _tree_reduce(op, xs):
    xs = list(xs)
    while len(xs) > 1:
        nxt = [op(xs[i], xs[i + 1]) for i in range(0, len(xs) - 1, 2)]
        if len(xs) % 2:
            nxt.append(xs[-1])
        xs = nxt
    return xs[0]


def _scores(emb, u_row):
    N, D = emb.shape
    BLK = 2000
    grid = N // BLK

    def body(e_ref, u_ref, o_ref):
        o_ref[...] = jnp.sum(e_ref[...] * u_ref[...], axis=1)[None, None, :]

    out = pl.pallas_call(
        body,
        grid=(grid,),
        in_specs=[
            pl.BlockSpec((BLK, D), lambda i: (i, 0)),
            pl.BlockSpec((1, D), lambda i: (0, 0)),
        ],
        out_specs=pl.BlockSpec((1, 1, BLK), lambda i: (i, 0, 0)),
        out_shape=jax.ShapeDtypeStruct((grid, 1, BLK), F32),
    )(emb, u_row)
    return out.reshape(N)


def _pack_table(emb_flat, N, D):
    NW = 32
    DW = D // 2
    ROWS_T = N // NW
    RCH = 125
    NCHP = ROWS_T // RCH
    NBP = 5
    mesh = plsc.VectorSubcoreMesh(core_axis_name="c", subcore_axis_name="s")
    cp = pltpu.CompilerParams()
    if "needs_layout_passes" in pltpu.CompilerParams.__dataclass_fields__:
        cp = dataclasses.replace(cp, needs_layout_passes=False)
    if "use_tc_tiling_on_sc" in pltpu.CompilerParams.__dataclass_fields__:
        cp = dataclasses.replace(cp, use_tc_tiling_on_sc=False)

    @functools.partial(
        pl.kernel,
        out_type=jax.ShapeDtypeStruct((N, DW), jnp.int32),
        mesh=mesh,
        compiler_params=cp,
        scratch_types=[
            pltpu.VMEM((NBP, RCH * D), F32),
            pltpu.VMEM((NBP, RCH, DW), jnp.int32),
            pltpu.SemaphoreType.DMA((NBP,)),
            pltpu.SemaphoreType.DMA((NBP,)),
        ],
    )
    def kern(src_hbm, dst_hbm, in_v, out_v, sem_i, sem_o):
        cid = lax.axis_index("c")
        sid = lax.axis_index("s")
        wid = sid * 2 + cid
        r0 = wid * ROWS_T
        ev = 2 * lax.iota(jnp.int32, 16)

        def _start_in(ch, j):
            pltpu.async_copy(
                src_hbm.at[pl.ds((r0 + ch * RCH) * D, RCH * D)],
                in_v.at[j], sem_i.at[j])

        for j in range(NBP):
            _start_in(j, j)

        @pl.loop(0, NCHP, step=NBP)
        def _go(c0):
            for j in range(NBP):
                ch = c0 + j
                pltpu.make_async_copy(src_hbm.at[pl.ds(0, RCH * D)],
                                      in_v.at[j], sem_i.at[j]).wait()

                @pl.when(c0 > 0)
                def _wait_out():
                    pltpu.make_async_copy(out_v.at[j],
                                          dst_hbm.at[pl.ds(0, RCH)],
                                          sem_o.at[j]).wait()

                @pl.loop(0, RCH)
                def _row(r):
                    for c in range(D // 32):
                        base = r * D + c * 32
                        a = plsc.load_gather(in_v.at[j], [base + ev])
                        b = plsc.load_gather(in_v.at[j], [base + ev + 1])
                        out_v[j, r, pl.ds(c * 16, 16)] = plsc.bitcast(
                            plsc.pack(a, b,
                                      format=plsc.PackFormat.INTERLEAVED),
                            jnp.int32)

                pltpu.async_copy(out_v.at[j],
                                 dst_hbm.at[pl.ds(r0 + ch * RCH, RCH)],
                                 sem_o.at[j])

                @pl.when(ch + NBP < NCHP)
                def _next_in():
                    _start_in(ch + NBP, j)

        for j in range(NBP):
            pltpu.make_async_copy(out_v.at[j], dst_hbm.at[pl.ds(0, RCH)],
                                  sem_o.at[j]).wait()

    return kern(emb_flat)


def _sc_agg(adj_flat, adj_t_flat, s, emb, F_PAD, FW_A, FW_B, K, D):
    FWMAX = max(FW_A, FW_B)
    FWSUM = FW_A + FW_B
    CH = 4
    GCH = 128
    mesh = plsc.VectorSubcoreMesh(core_axis_name="c", subcore_axis_name="s")
    NLANE = 16
    NSUB = D // NLANE
    cp = pltpu.CompilerParams()
    if "needs_layout_passes" in pltpu.CompilerParams.__dataclass_fields__:
        cp = dataclasses.replace(cp, needs_layout_passes=False)
    if "use_tc_tiling_on_sc" in pltpu.CompilerParams.__dataclass_fields__:
        cp = dataclasses.replace(cp, use_tc_tiling_on_sc=False)

    NBUF = 4

    BF16 = jnp.bfloat16
    DW = D // 2

    @functools.partial(
        pl.kernel,
        out_type=jax.ShapeDtypeStruct((F_PAD, DW), jnp.int32),
        mesh=mesh,
        compiler_params=cp,
        scratch_types=[
            pltpu.VMEM((FWMAX * K,), jnp.int32),
            pltpu.VMEM((FWMAX * K,), jnp.int32),
            pltpu.VMEM((FWMAX * K,), F32),
            pltpu.VMEM((FWMAX * K,), F32),
            pltpu.VMEM((NBUF, CH * K, DW), jnp.int32),
            pltpu.VMEM((FWMAX, DW), jnp.int32),
            pltpu.SemaphoreType.DMA,
            pltpu.SemaphoreType.DMA((NBUF,)),
        ],
    )
    def kern(adj_f_hbm, adj_t_hbm, s_hbm, emb_hbm, agg_hbm,
             adj_v, adjt_v, sg_v, w_v, rows_v, out_v,
             sem_m, sem_g):
        cid = lax.axis_index("c")
        sid = lax.axis_index("s")
        base_f = sid * FWSUM + cid * FW_A
        fw = jnp.where(cid == 0, FW_A, FW_B)

        pltpu.async_copy(adj_f_hbm.at[pl.ds(base_f * K, FWMAX * K)], adj_v,
                         sem_m)

        @pl.loop(0, K)
        def _adjt(k):
            pltpu.async_copy(
                adj_t_hbm.at[pl.ds(k * F_PAD + base_f, FWMAX)],
                adjt_v.at[pl.ds(k * FWMAX, FWMAX)],
                sem_m,
            )

        pltpu.make_async_copy(adj_f_hbm.at[pl.ds(0, FWMAX * K)], adj_v,
                              sem_m).wait()
        pltpu.make_async_copy(adj_t_hbm.at[pl.ds(0, FWMAX * K)], adjt_v,
                              sem_m).wait()

        def _start_gather(ch, j):
            pltpu.async_copy(
                emb_hbm.at[adj_v.at[pl.ds(ch * (CH * K), CH * K)]],
                rows_v.at[j], sem_g.at[j])

        for j in range(NBUF):
            _start_gather(j, j)

        @pl.loop(0, K)
        def _sg(k):
            @pl.loop(0, FWMAX // GCH)
            def _sgi(c):
                pltpu.async_copy(
                    s_hbm.at[adjt_v.at[pl.ds(k * FWMAX + c * GCH, GCH)]],
                    sg_v.at[pl.ds(k * FWMAX + c * GCH, GCH)],
                    sem_m,
                )

        @pl.loop(0, (K * FWMAX) // GCH)
        def _sgd(c):
            pltpu.make_async_copy(s_hbm.at[pl.ds(0, GCH)],
                                  sg_v.at[pl.ds(0, GCH)], sem_m).wait()

        @pl.loop(0, FWMAX // NLANE)
        def _smax(g):
            logits = []
            for k in range(K):
                off = k * FWMAX + g * NLANE
                a = adjt_v[pl.ds(off, NLANE)]
                sv = sg_v[pl.ds(off, NLANE)]
                logits.append(sv + jnp.where(a != 0, 0.0, -10000.0))
            mx = _tree_reduce(jnp.maximum, logits)
            es = [jnp.exp(l - mx) for l in logits]
            tot = _tree_reduce(jnp.add, es)
            inv = 1.0 / tot
            fidx = (lax.iota(jnp.int32, NLANE) + g * NLANE) * K
            for k in range(K):
                plsc.store_scatter(w_v, [fidx + k], es[k] * inv)

        NPAIR = DW // NLANE
        nch = fw // CH

        @pl.loop(0, nch, step=NBUF)
        def _acc(c0):
            for j in range(NBUF):
                ch = c0 + j
                f0 = ch * CH
                pltpu.make_async_copy(
                    emb_hbm.at[adj_v.at[pl.ds(0, CH * K)]],
                    rows_v.at[j], sem_g.at[j]).wait()

                for i in range(CH):
                    wva = w_v[pl.ds((f0 + i) * K, NLANE)]
                    wvb = w_v[pl.ds((f0 + i) * K + NLANE, NLANE)]
                    acc_a = [None] * NPAIR
                    acc_b = [None] * NPAIR
                    for k in range(K):
                        wk = wva[k] if k < NLANE else wvb[k - NLANE]
                        for c in range(NPAIR):
                            pk = plsc.bitcast(
                                rows_v[j, i * K + k,
                                       pl.ds(c * NLANE, NLANE)], BF16)
                            a, b = plsc.unpack(
                                pk, format=plsc.PackFormat.INTERLEAVED)
                            if k == 0:
                                acc_a[c] = wk * a
                                acc_b[c] = wk * b
                            else:
                                acc_a[c] = acc_a[c] + wk * a
                                acc_b[c] = acc_b[c] + wk * b
                    for c in range(NPAIR):
                        out_v[f0 + i, pl.ds(c * NLANE, NLANE)] = (
                            plsc.bitcast(
                                plsc.pack(acc_a[c], acc_b[c],
                                          format=plsc.PackFormat.INTERLEAVED),
                                jnp.int32))

                @pl.when(ch + NBUF < nch)
                def _next_gather():
                    _start_gather(ch + NBUF, j)

        @pl.when(cid == 0)
        def _store_a():
            pltpu.sync_copy(out_v.at[pl.ds(0, FW_A)],
                            agg_hbm.at[pl.ds(base_f, FW_A)])

        @pl.when(cid == 1)
        def _store_b():
            pltpu.sync_copy(out_v.at[pl.ds(0, FW_B)],
                            agg_hbm.at[pl.ds(base_f, FW_B)])

    return kern(adj_flat, adj_t_flat, s, emb)


def _gate(ef, ag, w1t, w2t, b_row):
    F, D = ef.shape
    BLK = 2000

    def body(ef_ref, ag_ref, w1_ref, w2_ref, b_ref, o_ref):
        e = ef_ref[...]
        a = ag_ref[...].astype(F32)
        g = (jnp.dot(e, w1_ref[...], preferred_element_type=F32)
             + jnp.dot(a, w2_ref[...], preferred_element_type=F32)
             + b_ref[...])
        g = jax.nn.sigmoid(g)
        o_ref[...] = g * e + (1.0 - g) * a

    return pl.pallas_call(
        body,
        grid=(F // BLK,),
        in_specs=[
            pl.BlockSpec((BLK, D), lambda i: (i, 0)),
            pl.BlockSpec((BLK, D), lambda i: (i, 0)),
            pl.BlockSpec((D, D), lambda i: (0, 0)),
            pl.BlockSpec((D, D), lambda i: (0, 0)),
            pl.BlockSpec((1, D), lambda i: (0, 0)),
        ],
        out_specs=pl.BlockSpec((BLK, D), lambda i: (i, 0)),
        out_shape=jax.ShapeDtypeStruct((F, D), F32),
    )(ef, ag, w1t, w2t, b_row)


def kernel(adjacency_fi, embedding_i, emb_f_weight, u, W_w, W_b):
    F, K = adjacency_fi.shape
    N, D = embedding_i.shape
    NW = 32
    FWSUM = 2 * (((F + NW - 1) // NW + 15) // 16 * 16)
    F_PAD = 16 * FWSUM
    FW_A = (FWSUM * 3 // 5) // 16 * 16
    FW_B = FWSUM - FW_A

    adj = adjacency_fi.astype(jnp.int32)
    adj = jnp.pad(adj, ((0, F_PAD - F), (0, 0)))
    adj_flat = adj.reshape(-1)
    adj_t_flat = adj.T.reshape(-1)

    s = _scores(embedding_i, u.reshape(1, D))
    emb_pk = _pack_table(embedding_i.reshape(N * D), N, D)
    agg_pk = _sc_agg(adj_flat, adj_t_flat, s, emb_pk, F_PAD, FW_A, FW_B,
                     K, D)
    agg = jax.lax.bitcast_convert_type(
        agg_pk, jnp.bfloat16).reshape(F_PAD, D)[:F]
    w1t = W_w[:, :D].T
    w2t = W_w[:, D:].T
    return _gate(emb_f_weight, agg, w1t, w2t, W_b.reshape(1, D))


# speedup vs baseline: 2.4524x; 1.0132x over previous
import dataclasses
import functools

import jax
import jax.numpy as jnp
from jax import lax
from jax.experimental import pallas as pl
from jax.experimental.pallas import tpu as pltpu
from jax.experimental.pallas import tpu_sc as plsc

F32 = jnp.float32


def _tree_reduce(op, xs):
    xs = list(xs)
    while len(xs) > 1:
        nxt = [op(xs[i], xs[i + 1]) for i in range(0, len(xs) - 1, 2)]
        if len(xs) % 2:
            nxt.append(xs[-1])
        xs = nxt
    return xs[0]


def _scores(emb, u_row):
    N, D = emb.shape
    BLK = 2000
    grid = N // BLK

    def body(e_ref, u_ref, o_ref):
        o_ref[...] = jnp.sum(e_ref[...] * u_ref[...], axis=1)[None, None, :]

    out = pl.pallas_call(
        body,
        grid=(grid,),
        in_specs=[
            pl.BlockSpec((BLK, D), lambda i: (i, 0)),
            pl.BlockSpec((1, D), lambda i: (0, 0)),
        ],
        out_specs=pl.BlockSpec((1, 1, BLK), lambda i: (i, 0, 0)),
        out_shape=jax.ShapeDtypeStruct((grid, 1, BLK), F32),
    )(emb, u_row)
    return out.reshape(N)


def _pack_table(emb_flat, N, D):
    NW = 32
    DW = D // 2
    ROWS_T = N // NW
    RCH = 125
    NCHP = ROWS_T // RCH
    NBP = 5
    mesh = plsc.VectorSubcoreMesh(core_axis_name="c", subcore_axis_name="s")
    cp = pltpu.CompilerParams()
    if "needs_layout_passes" in pltpu.CompilerParams.__dataclass_fields__:
        cp = dataclasses.replace(cp, needs_layout_passes=False)
    if "use_tc_tiling_on_sc" in pltpu.CompilerParams.__dataclass_fields__:
        cp = dataclasses.replace(cp, use_tc_tiling_on_sc=False)

    @functools.partial(
        pl.kernel,
        out_type=jax.ShapeDtypeStruct((N, DW), jnp.int32),
        mesh=mesh,
        compiler_params=cp,
        scratch_types=[
            pltpu.VMEM((NBP, RCH * D), F32),
            pltpu.VMEM((NBP, RCH, DW), jnp.int32),
            pltpu.SemaphoreType.DMA((NBP,)),
            pltpu.SemaphoreType.DMA((NBP,)),
        ],
    )
    def kern(src_hbm, dst_hbm, in_v, out_v, sem_i, sem_o):
        cid = lax.axis_index("c")
        sid = lax.axis_index("s")
        wid = sid * 2 + cid
        r0 = wid * ROWS_T
        ev = 2 * lax.iota(jnp.int32, 16)

        def _start_in(ch, j):
            pltpu.async_copy(
                src_hbm.at[pl.ds((r0 + ch * RCH) * D, RCH * D)],
                in_v.at[j], sem_i.at[j])

        for j in range(NBP):
            _start_in(j, j)

        @pl.loop(0, NCHP, step=NBP)
        def _go(c0):
            for j in range(NBP):
                ch = c0 + j
                pltpu.make_async_copy(src_hbm.at[pl.ds(0, RCH * D)],
                                      in_v.at[j], sem_i.at[j]).wait()

                @pl.when(c0 > 0)
                def _wait_out():
                    pltpu.make_async_copy(out_v.at[j],
                                          dst_hbm.at[pl.ds(0, RCH)],
                                          sem_o.at[j]).wait()

                @pl.loop(0, RCH)
                def _row(r):
                    for c in range(D // 32):
                        base = r * D + c * 32
                        a = plsc.load_gather(in_v.at[j], [base + ev])
                        b = plsc.load_gather(in_v.at[j], [base + ev + 1])
                        out_v[j, r, pl.ds(c * 16, 16)] = plsc.bitcast(
                            plsc.pack(a, b,
                                      format=plsc.PackFormat.INTERLEAVED),
                            jnp.int32)

                pltpu.async_copy(out_v.at[j],
                                 dst_hbm.at[pl.ds(r0 + ch * RCH, RCH)],
                                 sem_o.at[j])

                @pl.when(ch + NBP < NCHP)
                def _next_in():
                    _start_in(ch + NBP, j)

        for j in range(NBP):
            pltpu.make_async_copy(out_v.at[j], dst_hbm.at[pl.ds(0, RCH)],
                                  sem_o.at[j]).wait()

    return kern(emb_flat)


def _sc_agg(adj_flat, adj_t_flat, s, emb, F_PAD, FW_A, FW_B, K, D):
    FWMAX = max(FW_A, FW_B)
    FWSUM = FW_A + FW_B
    CH = 4
    GCH = 128
    mesh = plsc.VectorSubcoreMesh(core_axis_name="c", subcore_axis_name="s")
    NLANE = 16
    NSUB = D // NLANE
    cp = pltpu.CompilerParams()
    if "needs_layout_passes" in pltpu.CompilerParams.__dataclass_fields__:
        cp = dataclasses.replace(cp, needs_layout_passes=False)
    if "use_tc_tiling_on_sc" in pltpu.CompilerParams.__dataclass_fields__:
        cp = dataclasses.replace(cp, use_tc_tiling_on_sc=False)

    NBUF = 4

    BF16 = jnp.bfloat16
    DW = D // 2

    @functools.partial(
        pl.kernel,
        out_type=jax.ShapeDtypeStruct((F_PAD, DW), jnp.int32),
        mesh=mesh,
        compiler_params=cp,
        scratch_types=[
            pltpu.VMEM((FWMAX * K,), jnp.int32),
            pltpu.VMEM((FWMAX * K,), jnp.int32),
            pltpu.VMEM((FWMAX * K,), F32),
            pltpu.VMEM((FWMAX * K,), F32),
            pltpu.VMEM((NBUF, CH * K, DW), jnp.int32),
            pltpu.VMEM((FWMAX, DW), jnp.int32),
            pltpu.SemaphoreType.DMA,
            pltpu.SemaphoreType.DMA((NBUF,)),
        ],
    )
    def kern(adj_f_hbm, adj_t_hbm, s_hbm, emb_hbm, agg_hbm,
             adj_v, adjt_v, sg_v, w_v, rows_v, out_v,
             sem_m, sem_g):
        cid = lax.axis_index("c")
        sid = lax.axis_index("s")
        base_f = sid * FWSUM + cid * FW_A
        fw = jnp.where(cid == 0, FW_A, FW_B)

        pltpu.async_copy(adj_f_hbm.at[pl.ds(base_f * K, FWMAX * K)], adj_v,
                         sem_m)

        @pl.loop(0, K)
        def _adjt(k):
            pltpu.async_copy(
                adj_t_hbm.at[pl.ds(k * F_PAD + base_f, FWMAX)],
                adjt_v.at[pl.ds(k * FWMAX, FWMAX)],
                sem_m,
            )

        pltpu.make_async_copy(adj_f_hbm.at[pl.ds(0, FWMAX * K)], adj_v,
                              sem_m).wait()
        pltpu.make_async_copy(adj_t_hbm.at[pl.ds(0, FWMAX * K)], adjt_v,
                              sem_m).wait()

        def _start_gather(ch, j):
            pltpu.async_copy(
                emb_hbm.at[adj_v.at[pl.ds(ch * (CH * K), CH * K)]],
                rows_v.at[j], sem_g.at[j])

        for j in range(NBUF):
            _start_gather(j, j)

        fwc = fw // GCH

        @pl.loop(0, K)
        def _sg(k):
            @pl.loop(0, fwc)
            def _sgi(c):
                pltpu.async_copy(
                    s_hbm.at[adjt_v.at[pl.ds(k * FWMAX + c * GCH, GCH)]],
                    sg_v.at[pl.ds(k * FWMAX + c * GCH, GCH)],
                    sem_m,
                )

        @pl.loop(0, K * fwc)
        def _sgd(c):
            pltpu.make_async_copy(s_hbm.at[pl.ds(0, GCH)],
                                  sg_v.at[pl.ds(0, GCH)], sem_m).wait()

        @pl.loop(0, fw // NLANE)
        def _smax(g):
            logits = []
            for k in range(K):
                off = k * FWMAX + g * NLANE
                a = adjt_v[pl.ds(off, NLANE)]
                sv = sg_v[pl.ds(off, NLANE)]
                logits.append(sv + jnp.where(a != 0, 0.0, -10000.0))
            mx = _tree_reduce(jnp.maximum, logits)
            es = [jnp.exp(l - mx) for l in logits]
            tot = _tree_reduce(jnp.add, es)
            inv = 1.0 / tot
            fidx = (lax.iota(jnp.int32, NLANE) + g * NLANE) * K
            for k in range(K):
                plsc.store_scatter(w_v, [fidx + k], es[k] * inv)

        NPAIR = DW // NLANE
        nch = fw // CH

        @pl.loop(0, nch, step=NBUF)
        def _acc(c0):
            for j in range(NBUF):
                ch = c0 + j
                f0 = ch * CH
                pltpu.make_async_copy(
                    emb_hbm.at[adj_v.at[pl.ds(0, CH * K)]],
                    rows_v.at[j], sem_g.at[j]).wait()

                for i in range(CH):
                    wva = w_v[pl.ds((f0 + i) * K, NLANE)]
                    wvb = w_v[pl.ds((f0 + i) * K + NLANE, NLANE)]
                    acc_a = [None] * NPAIR
                    acc_b = [None] * NPAIR
                    for k in range(K):
                        wk = wva[k] if k < NLANE else wvb[k - NLANE]
                        for c in range(NPAIR):
                            pk = plsc.bitcast(
                                rows_v[j, i * K + k,
                                       pl.ds(c * NLANE, NLANE)], BF16)
                            a, b = plsc.unpack(
                                pk, format=plsc.PackFormat.INTERLEAVED)
                            if k == 0:
                                acc_a[c] = wk * a
                                acc_b[c] = wk * b
                            else:
                                acc_a[c] = acc_a[c] + wk * a
                                acc_b[c] = acc_b[c] + wk * b
                    for c in range(NPAIR):
                        out_v[f0 + i, pl.ds(c * NLANE, NLANE)] = (
                            plsc.bitcast(
                                plsc.pack(acc_a[c], acc_b[c],
                                          format=plsc.PackFormat.INTERLEAVED),
                                jnp.int32))

                @pl.when(ch + NBUF < nch)
                def _next_gather():
                    _start_gather(ch + NBUF, j)

        @pl.when(cid == 0)
        def _store_a():
            pltpu.sync_copy(out_v.at[pl.ds(0, FW_A)],
                            agg_hbm.at[pl.ds(base_f, FW_A)])

        @pl.when(cid == 1)
        def _store_b():
            pltpu.sync_copy(out_v.at[pl.ds(0, FW_B)],
                            agg_hbm.at[pl.ds(base_f, FW_B)])

    return kern(adj_flat, adj_t_flat, s, emb)


def _gate(ef, ag, w1t, w2t, b_row):
    F, D = ef.shape
    BLK = 2000

    def body(ef_ref, ag_ref, w1_ref, w2_ref, b_ref, o_ref):
        e = ef_ref[...]
        a = ag_ref[...].astype(F32)
        g = (jnp.dot(e, w1_ref[...], preferred_element_type=F32)
             + jnp.dot(a, w2_ref[...], preferred_element_type=F32)
             + b_ref[...])
        g = jax.nn.sigmoid(g)
        o_ref[...] = g * e + (1.0 - g) * a

    return pl.pallas_call(
        body,
        grid=(F // BLK,),
        in_specs=[
            pl.BlockSpec((BLK, D), lambda i: (i, 0)),
            pl.BlockSpec((BLK, D), lambda i: (i, 0)),
            pl.BlockSpec((D, D), lambda i: (0, 0)),
            pl.BlockSpec((D, D), lambda i: (0, 0)),
            pl.BlockSpec((1, D), lambda i: (0, 0)),
        ],
        out_specs=pl.BlockSpec((BLK, D), lambda i: (i, 0)),
        out_shape=jax.ShapeDtypeStruct((F, D), F32),
    )(ef, ag, w1t, w2t, b_row)


def kernel(adjacency_fi, embedding_i, emb_f_weight, u, W_w, W_b):
    F, K = adjacency_fi.shape
    N, D = embedding_i.shape
    NW = 32
    FWSUM = 2 * (((F + NW - 1) // NW + 15) // 16 * 16)
    F_PAD = 16 * FWSUM
    FW_A = (FWSUM * 3 // 5) // 16 * 16
    FW_B = FWSUM - FW_A

    adj = adjacency_fi.astype(jnp.int32)
    adj = jnp.pad(adj, ((0, F_PAD - F), (0, 0)))
    adj_flat = adj.reshape(-1)
    adj_t_flat = adj.T.reshape(-1)

    s = _scores(embedding_i, u.reshape(1, D))
    emb_pk = _pack_table(embedding_i.reshape(N * D), N, D)
    agg_pk = _sc_agg(adj_flat, adj_t_flat, s, emb_pk, F_PAD, FW_A, FW_B,
                     K, D)
    agg = jax.lax.bitcast_convert_type(
        agg_pk, jnp.bfloat16).reshape(F_PAD, D)[:F]
    w1t = W_w[:, :D].T
    w2t = W_w[:, D:].T
    return _gate(emb_f_weight, agg, w1t, w2t, W_b.reshape(1, D))


# speedup vs baseline: 2.4614x; 1.0037x over previous
import dataclasses
import functools

import jax
import jax.numpy as jnp
from jax import lax
from jax.experimental import pallas as pl
from jax.experimental.pallas import tpu as pltpu
from jax.experimental.pallas import tpu_sc as plsc

F32 = jnp.float32


def _tree_reduce(op, xs):
    xs = list(xs)
    while len(xs) > 1:
        nxt = [op(xs[i], xs[i + 1]) for i in range(0, len(xs) - 1, 2)]
        if len(xs) % 2:
            nxt.append(xs[-1])
        xs = nxt
    return xs[0]


def _scores(emb, u_row):
    N, D = emb.shape
    BLK = 2000
    grid = N // BLK

    def body(e_ref, u_ref, o_ref):
        o_ref[...] = jnp.sum(e_ref[...] * u_ref[...], axis=1)[None, None, :]

    out = pl.pallas_call(
        body,
        grid=(grid,),
        in_specs=[
            pl.BlockSpec((BLK, D), lambda i: (i, 0)),
            pl.BlockSpec((1, D), lambda i: (0, 0)),
        ],
        out_specs=pl.BlockSpec((1, 1, BLK), lambda i: (i, 0, 0)),
        out_shape=jax.ShapeDtypeStruct((grid, 1, BLK), F32),
    )(emb, u_row)
    return out.reshape(N)


def _pack_table(emb_flat, N, D):
    NW = 32
    DW = D // 2
    ROWS_T = N // NW
    RCH = 125
    NCHP = ROWS_T // RCH
    NBP = 5
    mesh = plsc.VectorSubcoreMesh(core_axis_name="c", subcore_axis_name="s")
    cp = pltpu.CompilerParams()
    if "needs_layout_passes" in pltpu.CompilerParams.__dataclass_fields__:
        cp = dataclasses.replace(cp, needs_layout_passes=False)
    if "use_tc_tiling_on_sc" in pltpu.CompilerParams.__dataclass_fields__:
        cp = dataclasses.replace(cp, use_tc_tiling_on_sc=False)

    @functools.partial(
        pl.kernel,
        out_type=jax.ShapeDtypeStruct((N, DW), jnp.int32),
        mesh=mesh,
        compiler_params=cp,
        scratch_types=[
            pltpu.VMEM((NBP, RCH * D), F32),
            pltpu.VMEM((NBP, RCH, DW), jnp.int32),
            pltpu.SemaphoreType.DMA((NBP,)),
            pltpu.SemaphoreType.DMA((NBP,)),
        ],
    )
    def kern(src_hbm, dst_hbm, in_v, out_v, sem_i, sem_o):
        cid = lax.axis_index("c")
        sid = lax.axis_index("s")
        wid = sid * 2 + cid
        r0 = wid * ROWS_T
        ev = 2 * lax.iota(jnp.int32, 16)

        def _start_in(ch, j):
            pltpu.async_copy(
                src_hbm.at[pl.ds((r0 + ch * RCH) * D, RCH * D)],
                in_v.at[j], sem_i.at[j])

        for j in range(NBP):
            _start_in(j, j)

        @pl.loop(0, NCHP, step=NBP)
        def _go(c0):
            for j in range(NBP):
                ch = c0 + j
                pltpu.make_async_copy(src_hbm.at[pl.ds(0, RCH * D)],
                                      in_v.at[j], sem_i.at[j]).wait()

                @pl.when(c0 > 0)
                def _wait_out():
                    pltpu.make_async_copy(out_v.at[j],
                                          dst_hbm.at[pl.ds(0, RCH)],
                                          sem_o.at[j]).wait()

                @pl.loop(0, RCH)
                def _row(r):
                    for c in range(D // 32):
                        base = r * D + c * 32
                        a = plsc.load_gather(in_v.at[j], [base + ev])
                        b = plsc.load_gather(in_v.at[j], [base + ev + 1])
                        out_v[j, r, pl.ds(c * 16, 16)] = plsc.bitcast(
                            plsc.pack(a, b,
                                      format=plsc.PackFormat.INTERLEAVED),
                            jnp.int32)

                pltpu.async_copy(out_v.at[j],
                                 dst_hbm.at[pl.ds(r0 + ch * RCH, RCH)],
                                 sem_o.at[j])

                @pl.when(ch + NBP < NCHP)
                def _next_in():
                    _start_in(ch + NBP, j)

        for j in range(NBP):
            pltpu.make_async_copy(out_v.at[j], dst_hbm.at[pl.ds(0, RCH)],
                                  sem_o.at[j]).wait()

    return kern(emb_flat)


def _sc_agg(adj_flat, adj_t_flat, s, emb, F_PAD, F_ALLOC, FW_A, FW_B, K, D):
    FWMAX = max(FW_A, FW_B)
    FWSUM = FW_A + FW_B
    CH = 4
    GCH = 128
    mesh = plsc.VectorSubcoreMesh(core_axis_name="c", subcore_axis_name="s")
    NLANE = 16
    NSUB = D // NLANE
    cp = pltpu.CompilerParams()
    if "needs_layout_passes" in pltpu.CompilerParams.__dataclass_fields__:
        cp = dataclasses.replace(cp, needs_layout_passes=False)
    if "use_tc_tiling_on_sc" in pltpu.CompilerParams.__dataclass_fields__:
        cp = dataclasses.replace(cp, use_tc_tiling_on_sc=False)

    NBUF = 4

    BF16 = jnp.bfloat16
    DW = D // 2

    @functools.partial(
        pl.kernel,
        out_type=jax.ShapeDtypeStruct((F_PAD, DW), jnp.int32),
        mesh=mesh,
        compiler_params=cp,
        scratch_types=[
            pltpu.VMEM((FWMAX * K,), jnp.int32),
            pltpu.VMEM((FWMAX * K,), jnp.int32),
            pltpu.VMEM((FWMAX * K,), F32),
            pltpu.VMEM((FWMAX * K,), F32),
            pltpu.VMEM((NBUF, CH * K, DW), jnp.int32),
            pltpu.VMEM((FWMAX, DW), jnp.int32),
            pltpu.SemaphoreType.DMA,
            pltpu.SemaphoreType.DMA((NBUF,)),
        ],
    )
    def kern(adj_f_hbm, adj_t_hbm, s_hbm, emb_hbm, agg_hbm,
             adj_v, adjt_v, sg_v, w_v, rows_v, out_v,
             sem_m, sem_g):
        cid = lax.axis_index("c")
        sid = lax.axis_index("s")
        base_f = sid * FWSUM + cid * FW_A
        fw = jnp.where(cid == 0, FW_A, FW_B)

        pltpu.async_copy(adj_f_hbm.at[pl.ds(base_f * K, FWMAX * K)], adj_v,
                         sem_m)

        @pl.loop(0, K)
        def _adjt(k):
            pltpu.async_copy(
                adj_t_hbm.at[pl.ds(k * F_ALLOC + base_f, FWMAX)],
                adjt_v.at[pl.ds(k * FWMAX, FWMAX)],
                sem_m,
            )

        pltpu.make_async_copy(adj_f_hbm.at[pl.ds(0, FWMAX * K)], adj_v,
                              sem_m).wait()
        pltpu.make_async_copy(adj_t_hbm.at[pl.ds(0, FWMAX * K)], adjt_v,
                              sem_m).wait()

        def _start_gather(ch, j):
            pltpu.async_copy(
                emb_hbm.at[adj_v.at[pl.ds(ch * (CH * K), CH * K)]],
                rows_v.at[j], sem_g.at[j])

        for j in range(NBUF):
            _start_gather(j, j)

        fwc = (fw + GCH - 1) // GCH

        @pl.loop(0, K)
        def _sg(k):
            @pl.loop(0, fwc)
            def _sgi(c):
                pltpu.async_copy(
                    s_hbm.at[adjt_v.at[pl.ds(k * FWMAX + c * GCH, GCH)]],
                    sg_v.at[pl.ds(k * FWMAX + c * GCH, GCH)],
                    sem_m,
                )

        @pl.loop(0, K * fwc)
        def _sgd(c):
            pltpu.make_async_copy(s_hbm.at[pl.ds(0, GCH)],
                                  sg_v.at[pl.ds(0, GCH)], sem_m).wait()

        @pl.loop(0, fw // NLANE)
        def _smax(g):
            logits = []
            for k in range(K):
                off = k * FWMAX + g * NLANE
                a = adjt_v[pl.ds(off, NLANE)]
                sv = sg_v[pl.ds(off, NLANE)]
                logits.append(sv + jnp.where(a != 0, 0.0, -10000.0))
            mx = _tree_reduce(jnp.maximum, logits)
            es = [jnp.exp(l - mx) for l in logits]
            tot = _tree_reduce(jnp.add, es)
            inv = 1.0 / tot
            fidx = (lax.iota(jnp.int32, NLANE) + g * NLANE) * K
            for k in range(K):
                plsc.store_scatter(w_v, [fidx + k], es[k] * inv)

        NPAIR = DW // NLANE
        nch = fw // CH

        @pl.loop(0, nch, step=NBUF)
        def _acc(c0):
            for j in range(NBUF):
                ch = c0 + j
                f0 = ch * CH
                pltpu.make_async_copy(
                    emb_hbm.at[adj_v.at[pl.ds(0, CH * K)]],
                    rows_v.at[j], sem_g.at[j]).wait()

                for i in range(CH):
                    wva = w_v[pl.ds((f0 + i) * K, NLANE)]
                    wvb = w_v[pl.ds((f0 + i) * K + NLANE, NLANE)]
                    acc_a = [None] * NPAIR
                    acc_b = [None] * NPAIR
                    for k in range(K):
                        wk = wva[k] if k < NLANE else wvb[k - NLANE]
                        for c in range(NPAIR):
                            pk = plsc.bitcast(
                                rows_v[j, i * K + k,
                                       pl.ds(c * NLANE, NLANE)], BF16)
                            a, b = plsc.unpack(
                                pk, format=plsc.PackFormat.INTERLEAVED)
                            if k == 0:
                                acc_a[c] = wk * a
                                acc_b[c] = wk * b
                            else:
                                acc_a[c] = acc_a[c] + wk * a
                                acc_b[c] = acc_b[c] + wk * b
                    for c in range(NPAIR):
                        out_v[f0 + i, pl.ds(c * NLANE, NLANE)] = (
                            plsc.bitcast(
                                plsc.pack(acc_a[c], acc_b[c],
                                          format=plsc.PackFormat.INTERLEAVED),
                                jnp.int32))

                @pl.when(ch + NBUF < nch)
                def _next_gather():
                    _start_gather(ch + NBUF, j)

        @pl.when(cid == 0)
        def _store_a():
            pltpu.sync_copy(out_v.at[pl.ds(0, FW_A)],
                            agg_hbm.at[pl.ds(base_f, FW_A)])

        @pl.when(cid == 1)
        def _store_b():
            pltpu.sync_copy(out_v.at[pl.ds(0, FW_B)],
                            agg_hbm.at[pl.ds(base_f, FW_B)])

    return kern(adj_flat, adj_t_flat, s, emb)


def _gate(ef, ag, w1t, w2t, b_row):
    F, D = ef.shape
    BLK = 2000

    def body(ef_ref, ag_ref, w1_ref, w2_ref, b_ref, o_ref):
        e = ef_ref[...]
        a = ag_ref[...].astype(F32)
        g = (jnp.dot(e, w1_ref[...], preferred_element_type=F32)
             + jnp.dot(a, w2_ref[...], preferred_element_type=F32)
             + b_ref[...])
        g = jax.nn.sigmoid(g)
        o_ref[...] = g * e + (1.0 - g) * a

    return pl.pallas_call(
        body,
        grid=(F // BLK,),
        in_specs=[
            pl.BlockSpec((BLK, D), lambda i: (i, 0)),
            pl.BlockSpec((BLK, D), lambda i: (i, 0)),
            pl.BlockSpec((D, D), lambda i: (0, 0)),
            pl.BlockSpec((D, D), lambda i: (0, 0)),
            pl.BlockSpec((1, D), lambda i: (0, 0)),
        ],
        out_specs=pl.BlockSpec((BLK, D), lambda i: (i, 0)),
        out_shape=jax.ShapeDtypeStruct((F, D), F32),
    )(ef, ag, w1t, w2t, b_row)


def kernel(adjacency_fi, embedding_i, emb_f_weight, u, W_w, W_b):
    F, K = adjacency_fi.shape
    N, D = embedding_i.shape
    NW = 32
    FWSUM = 2 * (((F + NW - 1) // NW + 15) // 16 * 16)
    F_PAD = 16 * FWSUM
    FW_A = (FWSUM * 13 // 20) // 16 * 16
    FW_B = FWSUM - FW_A

    FWMAX = max(FW_A, FW_B)
    F_ALLOC = F_PAD + FWMAX
    adj = adjacency_fi.astype(jnp.int32)
    adj = jnp.pad(adj, ((0, F_ALLOC - F), (0, 0)))
    adj_flat = adj.reshape(-1)
    adj_t_flat = adj.T.reshape(-1)

    s = _scores(embedding_i, u.reshape(1, D))
    emb_pk = _pack_table(embedding_i.reshape(N * D), N, D)
    agg_pk = _sc_agg(adj_flat, adj_t_flat, s, emb_pk, F_PAD, F_ALLOC,
                     FW_A, FW_B, K, D)
    agg = jax.lax.bitcast_convert_type(
        agg_pk, jnp.bfloat16).reshape(F_PAD, D)[:F]
    w1t = W_w[:, :D].T
    w2t = W_w[:, D:].T
    return _gate(emb_f_weight, agg, w1t, w2t, W_b.reshape(1, D))
